# Initial kernel scaffold; baseline (speedup 1.0000x reference)
import jax
import jax.numpy as jnp
from jax.experimental import pallas as pl


def kernel(x, edge_index, W1, b1, W2, b2):
    raise NotImplementedError("write your pallas kernel here")


# speedup vs baseline: 28.4087x; 28.4087x over previous
import functools

import jax
import jax.numpy as jnp
from jax import lax
from jax.experimental import pallas as pl
from jax.experimental.pallas import tpu as pltpu
from jax.experimental.pallas import tpu_sc as plsc

N = 10000
E = 320000
DF = 128
DH = 16
NCLS = 10

NC = 2
NS = 16
NW = NC * NS
EPT = E // NW
B = 80
NB = EPT // B
RPT = 624
TAIL0 = RPT * NS
TAILN = N - TAIL0

@functools.cache
def _sc_kernels():
    mesh = plsc.VectorSubcoreMesh(core_axis_name="c", subcore_axis_name="s")
    params = pltpu.CompilerParams(use_tc_tiling_on_sc=False)

    @functools.partial(
        pl.kernel,
        out_type=jax.ShapeDtypeStruct((NC, N, DH), jnp.float32),
        mesh=mesh,
        compiler_params=params,
        scratch_types=[
            pltpu.VMEM((NB, B), jnp.int32),
            pltpu.VMEM((B, DH), jnp.float32),
            pltpu.VMEM_SHARED((N, DH), jnp.float32),
        ],
    )
    def sc_degree(dst_hbm, zeros_hbm, ones_hbm, out_hbm, dst_v, ones_v, acc):
        c = lax.axis_index("c")
        s = lax.axis_index("s")
        wid = s * NC + c
        pltpu.sync_copy(zeros_hbm.at[pl.ds(s * RPT, RPT)], acc.at[pl.ds(s * RPT, RPT)])

        @pl.when(s == NS - 1)
        def _():
            pltpu.sync_copy(zeros_hbm.at[pl.ds(TAIL0, TAILN)], acc.at[pl.ds(TAIL0, TAILN)])

        pltpu.sync_copy(ones_hbm, ones_v)
        pltpu.sync_copy(dst_hbm.at[wid], dst_v)
        plsc.subcore_barrier()

        def body(j, carry):
            pltpu.sync_copy(ones_v, acc.at[dst_v.at[j]], add=True)
            return carry

        lax.fori_loop(0, NB, body, 0)
        plsc.subcore_barrier()
        pltpu.sync_copy(acc.at[pl.ds(s * RPT, RPT)], out_hbm.at[c, pl.ds(s * RPT, RPT)])

        @pl.when(s == NS - 1)
        def _():
            pltpu.sync_copy(acc.at[pl.ds(TAIL0, TAILN)], out_hbm.at[c, pl.ds(TAIL0, TAILN)])

    @functools.partial(
        pl.kernel,
        out_type=jax.ShapeDtypeStruct((NC, N, DH), jnp.float32),
        mesh=mesh,
        compiler_params=params,
        scratch_types=[
            pltpu.VMEM((NB, B), jnp.int32),
            pltpu.VMEM((NB, B), jnp.int32),
            pltpu.VMEM((B, DH), jnp.float32),
            pltpu.VMEM_SHARED((N, DH), jnp.float32),
            pltpu.SemaphoreType.DMA,
        ],
    )
    def sc_agg(g_hbm, src_hbm, dst_hbm, out_hbm, src_v, dst_v, msg_v, acc, sem):
        c = lax.axis_index("c")
        s = lax.axis_index("s")
        wid = s * NC + c
        pltpu.sync_copy(g_hbm.at[pl.ds(s * RPT, RPT)], acc.at[pl.ds(s * RPT, RPT)])

        @pl.when(s == NS - 1)
        def _():
            pltpu.sync_copy(g_hbm.at[pl.ds(TAIL0, TAILN)], acc.at[pl.ds(TAIL0, TAILN)])

        pltpu.sync_copy(src_hbm.at[wid], src_v)
        pltpu.sync_copy(dst_hbm.at[wid], dst_v)
        plsc.subcore_barrier()

        def body(j, carry):
            pltpu.async_copy(g_hbm.at[src_v.at[j]], msg_v, sem).wait()
            pltpu.sync_copy(msg_v, acc.at[dst_v.at[j]], add=True)
            return carry

        lax.fori_loop(0, NB, body, 0)
        plsc.subcore_barrier()
        pltpu.sync_copy(acc.at[pl.ds(s * RPT, RPT)], out_hbm.at[c, pl.ds(s * RPT, RPT)])

        @pl.when(s == NS - 1)
        def _():
            pltpu.sync_copy(acc.at[pl.ds(TAIL0, TAILN)], out_hbm.at[c, pl.ds(TAIL0, TAILN)])

    return sc_degree, sc_agg


_ROWS = 1000


def _tc1_body(degp_ref, x_ref, w1_ref, b1_ref, g_ref, dinv_ref):
    deg = degp_ref[0] + degp_ref[1] + 1.0
    dinv = lax.rsqrt(deg)
    h = jnp.dot(x_ref[...], w1_ref[...], preferred_element_type=jnp.float32)
    g_ref[...] = (h + b1_ref[...]) * dinv
    dinv_ref[...] = dinv


def _tc2_body(p_ref, g1_ref, dinv_ref, w2_ref, b2_ref, g2_ref):
    agg = p_ref[0] + p_ref[1] - g1_ref[...]
    out1 = jnp.maximum(dinv_ref[...] * agg, 0.0)
    h2 = jnp.dot(out1, w2_ref[...], preferred_element_type=jnp.float32)
    g2_ref[...] = (h2 + b2_ref[...]) * dinv_ref[...]


def _tc3_body(p_ref, g2_ref, dinv_ref, y_ref):
    out2 = dinv_ref[...] * (p_ref[0] + p_ref[1] - g2_ref[...])
    mask = lax.broadcasted_iota(jnp.int32, (_ROWS, DH), 1) < NCLS
    z = jnp.where(mask, out2, -jnp.inf)
    m = jnp.max(z, axis=1, keepdims=True)
    e = jnp.where(mask, jnp.exp(z - m), 0.0)
    y_ref[...] = e / jnp.sum(e, axis=1, keepdims=True)


def _row_spec(shape, idx):
    return pl.BlockSpec(shape, idx)


_grid = N // _ROWS

_tc1 = pl.pallas_call(
    _tc1_body,
    grid=(_grid,),
    in_specs=[
        pl.BlockSpec((NC, _ROWS, DH), lambda i: (0, i, 0)),
        pl.BlockSpec((_ROWS, DF), lambda i: (i, 0)),
        pl.BlockSpec((DF, DH), lambda i: (0, 0)),
        pl.BlockSpec((1, DH), lambda i: (0, 0)),
    ],
    out_specs=[
        pl.BlockSpec((_ROWS, DH), lambda i: (i, 0)),
        pl.BlockSpec((_ROWS, DH), lambda i: (i, 0)),
    ],
    out_shape=[
        jax.ShapeDtypeStruct((N, DH), jnp.float32),
        jax.ShapeDtypeStruct((N, DH), jnp.float32),
    ],
)

_tc2 = pl.pallas_call(
    _tc2_body,
    grid=(_grid,),
    in_specs=[
        pl.BlockSpec((NC, _ROWS, DH), lambda i: (0, i, 0)),
        pl.BlockSpec((_ROWS, DH), lambda i: (i, 0)),
        pl.BlockSpec((_ROWS, DH), lambda i: (i, 0)),
        pl.BlockSpec((DH, DH), lambda i: (0, 0)),
        pl.BlockSpec((1, DH), lambda i: (0, 0)),
    ],
    out_specs=pl.BlockSpec((_ROWS, DH), lambda i: (i, 0)),
    out_shape=jax.ShapeDtypeStruct((N, DH), jnp.float32),
)

_tc3 = pl.pallas_call(
    _tc3_body,
    grid=(_grid,),
    in_specs=[
        pl.BlockSpec((NC, _ROWS, DH), lambda i: (0, i, 0)),
        pl.BlockSpec((_ROWS, DH), lambda i: (i, 0)),
        pl.BlockSpec((_ROWS, DH), lambda i: (i, 0)),
    ],
    out_specs=pl.BlockSpec((_ROWS, DH), lambda i: (i, 0)),
    out_shape=jax.ShapeDtypeStruct((N, DH), jnp.float32),
)


@jax.jit
def kernel(x, edge_index, W1, b1, W2, b2):
    ei = edge_index.astype(jnp.int32)
    src3 = ei[0].reshape(NW, NB, B)
    dst3 = ei[1].reshape(NW, NB, B)

    zeros_nd = jnp.zeros((N, DH), jnp.float32)
    ones_b = jnp.ones((B, DH), jnp.float32)

    _sc_degree, _sc_agg = _sc_kernels()
    degp = _sc_degree(dst3, zeros_nd, ones_b)
    g1, dinv = _tc1(degp, x, W1, b1.reshape(1, DH))

    p1 = _sc_agg(g1, src3, dst3)
    w2p = jnp.zeros((DH, DH), jnp.float32).at[:, :NCLS].set(W2)
    b2p = jnp.zeros((1, DH), jnp.float32).at[0, :NCLS].set(b2)
    g2 = _tc2(p1, g1, dinv, w2p, b2p)

    p2 = _sc_agg(g2, src3, dst3)
    y = _tc3(p2, g2, dinv)
    return y[:, :NCLS]


# speedup vs baseline: 36.6781x; 1.2911x over previous
import functools

import jax
import jax.numpy as jnp
from jax import lax
from jax.experimental import pallas as pl
from jax.experimental.pallas import tpu as pltpu
from jax.experimental.pallas import tpu_sc as plsc

N = 10000
E = 320000
DF = 128
DH = 16
NCLS = 10

NC = 2
NS = 16
NW = NC * NS
EPT = E // NW
B = 125
NB = EPT // B
RPT = 624
TAIL0 = RPT * NS
TAILN = N - TAIL0

@functools.cache
def _sc_kernels():
    mesh = plsc.VectorSubcoreMesh(core_axis_name="c", subcore_axis_name="s")
    params = pltpu.CompilerParams(use_tc_tiling_on_sc=False)

    @functools.partial(
        pl.kernel,
        out_type=jax.ShapeDtypeStruct((NC, N, DH), jnp.float32),
        mesh=mesh,
        compiler_params=params,
        scratch_types=[
            pltpu.VMEM((NB, B), jnp.int32),
            pltpu.VMEM((B, DH), jnp.float32),
            pltpu.VMEM_SHARED((N, DH), jnp.float32),
        ],
    )
    def sc_degree(dst_hbm, zeros_hbm, ones_hbm, out_hbm, dst_v, ones_v, acc):
        c = lax.axis_index("c")
        s = lax.axis_index("s")
        wid = s * NC + c
        pltpu.sync_copy(zeros_hbm.at[pl.ds(s * RPT, RPT)], acc.at[pl.ds(s * RPT, RPT)])

        @pl.when(s == NS - 1)
        def _():
            pltpu.sync_copy(zeros_hbm.at[pl.ds(TAIL0, TAILN)], acc.at[pl.ds(TAIL0, TAILN)])

        pltpu.sync_copy(ones_hbm, ones_v)
        pltpu.sync_copy(dst_hbm.at[wid], dst_v)
        plsc.subcore_barrier()

        def body(j, carry):
            pltpu.sync_copy(ones_v, acc.at[dst_v.at[j]], add=True)
            return carry

        lax.fori_loop(0, NB, body, 0)
        plsc.subcore_barrier()
        pltpu.sync_copy(acc.at[pl.ds(s * RPT, RPT)], out_hbm.at[c, pl.ds(s * RPT, RPT)])

        @pl.when(s == NS - 1)
        def _():
            pltpu.sync_copy(acc.at[pl.ds(TAIL0, TAILN)], out_hbm.at[c, pl.ds(TAIL0, TAILN)])

    @functools.partial(
        pl.kernel,
        out_type=jax.ShapeDtypeStruct((NC, N, DH), jnp.float32),
        mesh=mesh,
        compiler_params=params,
        scratch_types=[
            pltpu.VMEM((NB, B), jnp.int32),
            pltpu.VMEM((NB, B), jnp.int32),
            pltpu.VMEM((B, DH), jnp.float32),
            pltpu.VMEM((B, DH), jnp.float32),
            pltpu.VMEM_SHARED((N, DH), jnp.float32),
            pltpu.SemaphoreType.DMA,
            pltpu.SemaphoreType.DMA,
        ],
    )
    def sc_agg(g_hbm, src_hbm, dst_hbm, out_hbm, src_v, dst_v, msg0, msg1, acc, sem0, sem1):
        c = lax.axis_index("c")
        s = lax.axis_index("s")
        wid = s * NC + c
        pltpu.sync_copy(g_hbm.at[pl.ds(s * RPT, RPT)], acc.at[pl.ds(s * RPT, RPT)])

        @pl.when(s == NS - 1)
        def _():
            pltpu.sync_copy(g_hbm.at[pl.ds(TAIL0, TAILN)], acc.at[pl.ds(TAIL0, TAILN)])

        pltpu.sync_copy(src_hbm.at[wid], src_v)
        pltpu.sync_copy(dst_hbm.at[wid], dst_v)
        plsc.subcore_barrier()

        pltpu.async_copy(g_hbm.at[src_v.at[0]], msg0, sem0)
        ngrp = NB // 2

        def body(g, carry):
            j0 = 2 * g
            pltpu.make_async_copy(g_hbm.at[src_v.at[j0]], msg0, sem0).wait()
            pltpu.async_copy(g_hbm.at[src_v.at[j0 + 1]], msg1, sem1)
            pltpu.sync_copy(msg0, acc.at[dst_v.at[j0]], add=True)
            pltpu.make_async_copy(g_hbm.at[src_v.at[j0 + 1]], msg1, sem1).wait()

            @pl.when(g < ngrp - 1)
            def _():
                pltpu.async_copy(g_hbm.at[src_v.at[j0 + 2]], msg0, sem0)

            pltpu.sync_copy(msg1, acc.at[dst_v.at[j0 + 1]], add=True)
            return carry

        lax.fori_loop(0, ngrp, body, 0)
        plsc.subcore_barrier()
        pltpu.sync_copy(acc.at[pl.ds(s * RPT, RPT)], out_hbm.at[c, pl.ds(s * RPT, RPT)])

        @pl.when(s == NS - 1)
        def _():
            pltpu.sync_copy(acc.at[pl.ds(TAIL0, TAILN)], out_hbm.at[c, pl.ds(TAIL0, TAILN)])

    return sc_degree, sc_agg


_ROWS = 1000


def _tc1_body(degp_ref, x_ref, w1_ref, b1_ref, g_ref, dinv_ref):
    deg = degp_ref[0] + degp_ref[1] + 1.0
    dinv = lax.rsqrt(deg)
    h = jnp.dot(x_ref[...], w1_ref[...], preferred_element_type=jnp.float32)
    g_ref[...] = (h + b1_ref[...]) * dinv
    dinv_ref[...] = dinv


def _tc2_body(p_ref, g1_ref, dinv_ref, w2_ref, b2_ref, g2_ref):
    agg = p_ref[0] + p_ref[1] - g1_ref[...]
    out1 = jnp.maximum(dinv_ref[...] * agg, 0.0)
    h2 = jnp.dot(out1, w2_ref[...], preferred_element_type=jnp.float32)
    g2_ref[...] = (h2 + b2_ref[...]) * dinv_ref[...]


def _tc3_body(p_ref, g2_ref, dinv_ref, y_ref):
    out2 = dinv_ref[...] * (p_ref[0] + p_ref[1] - g2_ref[...])
    mask = lax.broadcasted_iota(jnp.int32, (_ROWS, DH), 1) < NCLS
    z = jnp.where(mask, out2, -jnp.inf)
    m = jnp.max(z, axis=1, keepdims=True)
    e = jnp.where(mask, jnp.exp(z - m), 0.0)
    y_ref[...] = e / jnp.sum(e, axis=1, keepdims=True)


def _row_spec(shape, idx):
    return pl.BlockSpec(shape, idx)


_grid = N // _ROWS

_tc1 = pl.pallas_call(
    _tc1_body,
    grid=(_grid,),
    in_specs=[
        pl.BlockSpec((NC, _ROWS, DH), lambda i: (0, i, 0)),
        pl.BlockSpec((_ROWS, DF), lambda i: (i, 0)),
        pl.BlockSpec((DF, DH), lambda i: (0, 0)),
        pl.BlockSpec((1, DH), lambda i: (0, 0)),
    ],
    out_specs=[
        pl.BlockSpec((_ROWS, DH), lambda i: (i, 0)),
        pl.BlockSpec((_ROWS, DH), lambda i: (i, 0)),
    ],
    out_shape=[
        jax.ShapeDtypeStruct((N, DH), jnp.float32),
        jax.ShapeDtypeStruct((N, DH), jnp.float32),
    ],
)

_tc2 = pl.pallas_call(
    _tc2_body,
    grid=(_grid,),
    in_specs=[
        pl.BlockSpec((NC, _ROWS, DH), lambda i: (0, i, 0)),
        pl.BlockSpec((_ROWS, DH), lambda i: (i, 0)),
        pl.BlockSpec((_ROWS, DH), lambda i: (i, 0)),
        pl.BlockSpec((DH, DH), lambda i: (0, 0)),
        pl.BlockSpec((1, DH), lambda i: (0, 0)),
    ],
    out_specs=pl.BlockSpec((_ROWS, DH), lambda i: (i, 0)),
    out_shape=jax.ShapeDtypeStruct((N, DH), jnp.float32),
)

_tc3 = pl.pallas_call(
    _tc3_body,
    grid=(_grid,),
    in_specs=[
        pl.BlockSpec((NC, _ROWS, DH), lambda i: (0, i, 0)),
        pl.BlockSpec((_ROWS, DH), lambda i: (i, 0)),
        pl.BlockSpec((_ROWS, DH), lambda i: (i, 0)),
    ],
    out_specs=pl.BlockSpec((_ROWS, DH), lambda i: (i, 0)),
    out_shape=jax.ShapeDtypeStruct((N, DH), jnp.float32),
)


@jax.jit
def kernel(x, edge_index, W1, b1, W2, b2):
    ei = edge_index.astype(jnp.int32)
    src3 = ei[0].reshape(NW, NB, B)
    dst3 = ei[1].reshape(NW, NB, B)

    zeros_nd = jnp.zeros((N, DH), jnp.float32)
    ones_b = jnp.ones((B, DH), jnp.float32)

    _sc_degree, _sc_agg = _sc_kernels()
    degp = _sc_degree(dst3, zeros_nd, ones_b)
    g1, dinv = _tc1(degp, x, W1, b1.reshape(1, DH))

    p1 = _sc_agg(g1, src3, dst3)
    w2p = jnp.zeros((DH, DH), jnp.float32).at[:, :NCLS].set(W2)
    b2p = jnp.zeros((1, DH), jnp.float32).at[0, :NCLS].set(b2)
    g2 = _tc2(p1, g1, dinv, w2p, b2p)

    p2 = _sc_agg(g2, src3, dst3)
    y = _tc3(p2, g2, dinv)
    return y[:, :NCLS]


# speedup vs baseline: 44.7434x; 1.2199x over previous
import functools

import jax
import jax.numpy as jnp
from jax import lax
from jax.experimental import pallas as pl
from jax.experimental.pallas import tpu as pltpu
from jax.experimental.pallas import tpu_sc as plsc

N = 10000
E = 320000
DF = 128
DH = 16
NCLS = 10

NC = 2
NS = 16
NW = NC * NS
EPT = E // NW
B = 125
NB = EPT // B
RPT = 624
TAIL0 = RPT * NS
TAILN = N - TAIL0

@functools.cache
def _sc_kernels():
    mesh = plsc.VectorSubcoreMesh(core_axis_name="c", subcore_axis_name="s")
    params = pltpu.CompilerParams(use_tc_tiling_on_sc=False)

    @functools.partial(
        pl.kernel,
        out_type=jax.ShapeDtypeStruct((NC, N, DH), jnp.float32),
        mesh=mesh,
        compiler_params=params,
        scratch_types=[
            pltpu.VMEM((NB, B), jnp.int32),
            pltpu.VMEM((B, DH), jnp.float32),
            pltpu.VMEM_SHARED((N, DH), jnp.float32),
        ],
    )
    def sc_degree(dst_hbm, zeros_hbm, ones_hbm, out_hbm, dst_v, ones_v, acc):
        c = lax.axis_index("c")
        s = lax.axis_index("s")
        wid = s * NC + c
        pltpu.sync_copy(zeros_hbm.at[pl.ds(s * RPT, RPT)], acc.at[pl.ds(s * RPT, RPT)])

        @pl.when(s == NS - 1)
        def _():
            pltpu.sync_copy(zeros_hbm.at[pl.ds(TAIL0, TAILN)], acc.at[pl.ds(TAIL0, TAILN)])

        pltpu.sync_copy(ones_hbm, ones_v)
        pltpu.sync_copy(dst_hbm.at[wid], dst_v)
        plsc.subcore_barrier()

        def body(j, carry):
            pltpu.sync_copy(ones_v, acc.at[dst_v.at[j]], add=True)
            return carry

        lax.fori_loop(0, NB, body, 0)
        plsc.subcore_barrier()
        pltpu.sync_copy(acc.at[pl.ds(s * RPT, RPT)], out_hbm.at[c, pl.ds(s * RPT, RPT)])

        @pl.when(s == NS - 1)
        def _():
            pltpu.sync_copy(acc.at[pl.ds(TAIL0, TAILN)], out_hbm.at[c, pl.ds(TAIL0, TAILN)])

    @functools.partial(
        pl.kernel,
        out_type=jax.ShapeDtypeStruct((NC, N, DH), jnp.float32),
        mesh=mesh,
        compiler_params=params,
        scratch_types=[
            pltpu.VMEM((NB, B), jnp.int32),
            pltpu.VMEM((NB, B), jnp.int32),
            pltpu.VMEM((B, DH), jnp.float32),
            pltpu.VMEM((B, DH), jnp.float32),
            pltpu.VMEM_SHARED((N, DH), jnp.float32),
            pltpu.SemaphoreType.DMA,
            pltpu.SemaphoreType.DMA,
        ],
    )
    def sc_agg(g_hbm, src_hbm, dst_hbm, out_hbm, src_v, dst_v, msg0, msg1, acc, sem0, sem1):
        c = lax.axis_index("c")
        s = lax.axis_index("s")
        wid = s * NC + c
        pltpu.sync_copy(g_hbm.at[pl.ds(s * RPT, RPT)], acc.at[pl.ds(s * RPT, RPT)])

        @pl.when(s == NS - 1)
        def _():
            pltpu.sync_copy(g_hbm.at[pl.ds(TAIL0, TAILN)], acc.at[pl.ds(TAIL0, TAILN)])

        pltpu.sync_copy(src_hbm.at[wid], src_v)
        pltpu.sync_copy(dst_hbm.at[wid], dst_v)
        plsc.subcore_barrier()

        pltpu.async_copy(g_hbm.at[src_v.at[0]], msg0, sem0)
        ngrp = NB // 2

        def body(g, carry):
            j0 = 2 * g
            pltpu.make_async_copy(g_hbm.at[src_v.at[j0]], msg0, sem0).wait()
            pltpu.async_copy(g_hbm.at[src_v.at[j0 + 1]], msg1, sem1)
            pltpu.sync_copy(msg0, acc.at[dst_v.at[j0]], add=True)
            pltpu.make_async_copy(g_hbm.at[src_v.at[j0 + 1]], msg1, sem1).wait()

            @pl.when(g < ngrp - 1)
            def _():
                pltpu.async_copy(g_hbm.at[src_v.at[j0 + 2]], msg0, sem0)

            pltpu.sync_copy(msg1, acc.at[dst_v.at[j0 + 1]], add=True)
            return carry

        lax.fori_loop(0, ngrp, body, 0)
        plsc.subcore_barrier()
        pltpu.sync_copy(acc.at[pl.ds(s * RPT, RPT)], out_hbm.at[c, pl.ds(s * RPT, RPT)])

        @pl.when(s == NS - 1)
        def _():
            pltpu.sync_copy(acc.at[pl.ds(TAIL0, TAILN)], out_hbm.at[c, pl.ds(TAIL0, TAILN)])

    return sc_degree, sc_agg


NP = N // 8
GP = 8


def _tc1_body(degp_ref, x3_ref, w1_ref, b1_ref, g_ref, dinv_ref):
    deg = degp_ref[0] + degp_ref[1] + 1.0
    dinv = lax.rsqrt(deg)
    w1 = w1_ref[...]
    h = jnp.concatenate(
        [
            jnp.dot(x3_ref[:, j, :], w1, preferred_element_type=jnp.float32)
            for j in range(GP)
        ],
        axis=1,
    )
    g_ref[...] = (h + b1_ref[...]) * dinv
    dinv_ref[...] = dinv


def _tc2_body(p_ref, g1_ref, dinv_ref, m2_ref, b2_ref, g2_ref):
    agg = p_ref[0] + p_ref[1] - g1_ref[...]
    out1 = jnp.maximum(dinv_ref[...] * agg, 0.0)
    h2 = jnp.dot(out1, m2_ref[...], preferred_element_type=jnp.float32)
    g2_ref[...] = (h2 + b2_ref[...]) * dinv_ref[...]


def _tc3_body(p_ref, g2_ref, dinv_ref, s0_ref, y_ref):
    out2 = dinv_ref[...] * (p_ref[0] + p_ref[1] - g2_ref[...])
    lane = lax.broadcasted_iota(jnp.int32, (NP, GP * DH), 1)
    mask = (lane % DH) < NCLS
    zm = jnp.where(mask, out2, 0.0)
    s0 = s0_ref[...]
    mean = jnp.dot(zm, s0, preferred_element_type=jnp.float32) * (1.0 / NCLS)
    e = jnp.where(mask, jnp.exp(out2 - mean), 0.0)
    tot = jnp.dot(e, s0, preferred_element_type=jnp.float32)
    y_ref[...] = e / tot


_full = lambda shape: pl.BlockSpec(shape, lambda: tuple(0 for _ in shape))

_tc1 = pl.pallas_call(
    _tc1_body,
    in_specs=[
        _full((NC, NP, GP * DH)),
        _full((NP, GP, DF)),
        _full((DF, DH)),
        _full((1, GP * DH)),
    ],
    out_specs=[_full((NP, GP * DH)), _full((NP, GP * DH))],
    out_shape=[
        jax.ShapeDtypeStruct((NP, GP * DH), jnp.float32),
        jax.ShapeDtypeStruct((NP, GP * DH), jnp.float32),
    ],
)

_tc2 = pl.pallas_call(
    _tc2_body,
    in_specs=[
        _full((NC, NP, GP * DH)),
        _full((NP, GP * DH)),
        _full((NP, GP * DH)),
        _full((GP * DH, GP * DH)),
        _full((1, GP * DH)),
    ],
    out_specs=_full((NP, GP * DH)),
    out_shape=jax.ShapeDtypeStruct((NP, GP * DH), jnp.float32),
)

_tc3 = pl.pallas_call(
    _tc3_body,
    in_specs=[
        _full((NC, NP, GP * DH)),
        _full((NP, GP * DH)),
        _full((NP, GP * DH)),
        _full((GP * DH, GP * DH)),
    ],
    out_specs=_full((NP, GP * DH)),
    out_shape=jax.ShapeDtypeStruct((NP, GP * DH), jnp.float32),
)


@jax.jit
def kernel(x, edge_index, W1, b1, W2, b2):
    ei = edge_index.astype(jnp.int32)
    src3 = ei[0].reshape(NW, NB, B)
    dst3 = ei[1].reshape(NW, NB, B)

    zeros_nd = jnp.zeros((N, DH), jnp.float32)
    ones_b = jnp.ones((B, DH), jnp.float32)

    _sc_degree, _sc_agg = _sc_kernels()
    degp = _sc_degree(dst3, zeros_nd, ones_b)

    x3 = x.reshape(NP, GP, DF)
    b1t = jnp.tile(b1, GP).reshape(1, GP * DH)
    g1_p, dinv_p = _tc1(degp.reshape(NC, NP, GP * DH), x3, W1, b1t)

    p1 = _sc_agg(g1_p.reshape(N, DH), src3, dst3)
    w2p = jnp.zeros((DH, DH), jnp.float32).at[:, :NCLS].set(W2)
    m2 = jnp.kron(jnp.eye(GP, dtype=jnp.float32), w2p)
    b2p = jnp.zeros((DH,), jnp.float32).at[:NCLS].set(b2)
    b2t = jnp.tile(b2p, GP).reshape(1, GP * DH)
    g2_p = _tc2(p1.reshape(NC, NP, GP * DH), g1_p, dinv_p, m2, b2t)

    p2 = _sc_agg(g2_p.reshape(N, DH), src3, dst3)
    s0 = jnp.kron(jnp.eye(GP, dtype=jnp.float32), jnp.ones((DH, DH), jnp.float32))
    y_p = _tc3(p2.reshape(NC, NP, GP * DH), g2_p, dinv_p, s0)
    return y_p.reshape(N, DH)[:, :NCLS]


# speedup vs baseline: 68.1155x; 1.5224x over previous
import functools

import jax
import jax.numpy as jnp
from jax import lax
from jax.experimental import pallas as pl
from jax.experimental.pallas import tpu as pltpu
from jax.experimental.pallas import tpu_sc as plsc

N = 10000
E = 320000
DF = 128
DH = 16
NCLS = 10

NC = 2
NS = 16
NW = NC * NS
EPT = E // NW
B = 125
NB = EPT // B
RPT = 624
TAIL0 = RPT * NS
TAILN = N - TAIL0

@functools.cache
def _sc_kernels():
    mesh = plsc.VectorSubcoreMesh(core_axis_name="c", subcore_axis_name="s")
    params = pltpu.CompilerParams(use_tc_tiling_on_sc=False)

    @functools.partial(
        pl.kernel,
        out_type=jax.ShapeDtypeStruct((NC, N, DH), jnp.float32),
        mesh=mesh,
        compiler_params=params,
        scratch_types=[
            pltpu.VMEM((NB, B), jnp.int32),
            pltpu.VMEM((B, DH), jnp.float32),
            pltpu.VMEM_SHARED((N, DH), jnp.float32),
        ],
    )
    def sc_degree(dst_hbm, zeros_hbm, ones_hbm, out_hbm, dst_v, ones_v, acc):
        c = lax.axis_index("c")
        s = lax.axis_index("s")
        wid = s * NC + c
        pltpu.sync_copy(zeros_hbm.at[pl.ds(s * RPT, RPT)], acc.at[pl.ds(s * RPT, RPT)])

        @pl.when(s == NS - 1)
        def _():
            pltpu.sync_copy(zeros_hbm.at[pl.ds(TAIL0, TAILN)], acc.at[pl.ds(TAIL0, TAILN)])

        pltpu.sync_copy(ones_hbm, ones_v)
        pltpu.sync_copy(dst_hbm.at[wid], dst_v)
        plsc.subcore_barrier()

        def body(j, carry):
            pltpu.sync_copy(ones_v, acc.at[dst_v.at[j]], add=True)
            return carry

        lax.fori_loop(0, NB, body, 0)
        plsc.subcore_barrier()
        pltpu.sync_copy(acc.at[pl.ds(s * RPT, RPT)], out_hbm.at[c, pl.ds(s * RPT, RPT)])

        @pl.when(s == NS - 1)
        def _():
            pltpu.sync_copy(acc.at[pl.ds(TAIL0, TAILN)], out_hbm.at[c, pl.ds(TAIL0, TAILN)])

    @functools.partial(
        pl.kernel,
        out_type=jax.ShapeDtypeStruct((NC, N, DH), jnp.float32),
        mesh=mesh,
        compiler_params=params,
        scratch_types=[
            pltpu.VMEM((NB, B), jnp.int32),
            pltpu.VMEM((NB, B), jnp.int32),
            pltpu.VMEM((4, B, DH), jnp.float32),
            pltpu.VMEM_SHARED((N, DH), jnp.float32),
            [pltpu.SemaphoreType.DMA] * 4,
        ],
    )
    def sc_agg(g_hbm, src_hbm, dst_hbm, out_hbm, src_v, dst_v, msg, acc, sems):
        c = lax.axis_index("c")
        s = lax.axis_index("s")
        wid = s * NC + c
        pltpu.sync_copy(g_hbm.at[pl.ds(s * RPT, RPT)], acc.at[pl.ds(s * RPT, RPT)])

        @pl.when(s == NS - 1)
        def _():
            pltpu.sync_copy(g_hbm.at[pl.ds(TAIL0, TAILN)], acc.at[pl.ds(TAIL0, TAILN)])

        pltpu.sync_copy(src_hbm.at[wid], src_v)
        pltpu.sync_copy(dst_hbm.at[wid], dst_v)
        plsc.subcore_barrier()

        for k in range(3):
            pltpu.async_copy(g_hbm.at[src_v.at[k]], msg.at[k], sems[k])
        ngrp = NB // 4

        def body(g, carry):
            j0 = 4 * g
            for k in range(4):
                j = j0 + k
                kn = (k + 3) % 4
                pltpu.make_async_copy(g_hbm.at[src_v.at[j]], msg.at[k], sems[k]).wait()

                @pl.when(j + 3 < NB)
                def _():
                    pltpu.async_copy(g_hbm.at[src_v.at[j + 3]], msg.at[kn], sems[kn])

                pltpu.sync_copy(msg.at[k], acc.at[dst_v.at[j]], add=True)
            return carry

        lax.fori_loop(0, ngrp, body, 0)
        plsc.subcore_barrier()
        pltpu.sync_copy(acc.at[pl.ds(s * RPT, RPT)], out_hbm.at[c, pl.ds(s * RPT, RPT)])

        @pl.when(s == NS - 1)
        def _():
            pltpu.sync_copy(acc.at[pl.ds(TAIL0, TAILN)], out_hbm.at[c, pl.ds(TAIL0, TAILN)])

    return sc_degree, sc_agg


NP = N // 8
GP = 8


def _tc1_body(degp_ref, x3_ref, w1_ref, b1_ref, g_ref, dinv_ref):
    deg = degp_ref[0] + degp_ref[1] + 1.0
    dinv = lax.rsqrt(deg)
    w1 = w1_ref[...]
    h = jnp.concatenate(
        [
            jnp.dot(x3_ref[:, j, :], w1, preferred_element_type=jnp.float32)
            for j in range(GP)
        ],
        axis=1,
    )
    g_ref[...] = (h + b1_ref[...]) * dinv
    dinv_ref[...] = dinv


def _tc2_body(p_ref, g1_ref, dinv_ref, m2_ref, b2_ref, g2_ref):
    agg = p_ref[0] + p_ref[1] - g1_ref[...]
    out1 = jnp.maximum(dinv_ref[...] * agg, 0.0)
    h2 = jnp.dot(out1, m2_ref[...], preferred_element_type=jnp.float32)
    g2_ref[...] = (h2 + b2_ref[...]) * dinv_ref[...]


def _tc3_body(p_ref, g2_ref, dinv_ref, s0_ref, y_ref):
    out2 = dinv_ref[...] * (p_ref[0] + p_ref[1] - g2_ref[...])
    lane = lax.broadcasted_iota(jnp.int32, (NP, GP * DH), 1)
    mask = (lane % DH) < NCLS
    zm = jnp.where(mask, out2, 0.0)
    s0 = s0_ref[...]
    mean = jnp.dot(zm, s0, preferred_element_type=jnp.float32) * (1.0 / NCLS)
    e = jnp.where(mask, jnp.exp(out2 - mean), 0.0)
    tot = jnp.dot(e, s0, preferred_element_type=jnp.float32)
    y_ref[...] = e / tot


_full = lambda shape: pl.BlockSpec(shape, lambda: tuple(0 for _ in shape))

_tc1 = pl.pallas_call(
    _tc1_body,
    in_specs=[
        _full((NC, NP, GP * DH)),
        _full((NP, GP, DF)),
        _full((DF, DH)),
        _full((1, GP * DH)),
    ],
    out_specs=[_full((NP, GP * DH)), _full((NP, GP * DH))],
    out_shape=[
        jax.ShapeDtypeStruct((NP, GP * DH), jnp.float32),
        jax.ShapeDtypeStruct((NP, GP * DH), jnp.float32),
    ],
)

_tc2 = pl.pallas_call(
    _tc2_body,
    in_specs=[
        _full((NC, NP, GP * DH)),
        _full((NP, GP * DH)),
        _full((NP, GP * DH)),
        _full((GP * DH, GP * DH)),
        _full((1, GP * DH)),
    ],
    out_specs=_full((NP, GP * DH)),
    out_shape=jax.ShapeDtypeStruct((NP, GP * DH), jnp.float32),
)

_tc3 = pl.pallas_call(
    _tc3_body,
    in_specs=[
        _full((NC, NP, GP * DH)),
        _full((NP, GP * DH)),
        _full((NP, GP * DH)),
        _full((GP * DH, GP * DH)),
    ],
    out_specs=_full((NP, GP * DH)),
    out_shape=jax.ShapeDtypeStruct((NP, GP * DH), jnp.float32),
)


@jax.jit
def kernel(x, edge_index, W1, b1, W2, b2):
    ei = edge_index.astype(jnp.int32)
    src3 = ei[0].reshape(NW, NB, B)
    dst3 = ei[1].reshape(NW, NB, B)

    zeros_nd = jnp.zeros((N, DH), jnp.float32)
    ones_b = jnp.ones((B, DH), jnp.float32)

    _sc_degree, _sc_agg = _sc_kernels()
    degp = _sc_degree(dst3, zeros_nd, ones_b)

    x3 = x.reshape(NP, GP, DF)
    b1t = jnp.tile(b1, GP).reshape(1, GP * DH)
    g1_p, dinv_p = _tc1(degp.reshape(NC, NP, GP * DH), x3, W1, b1t)

    p1 = _sc_agg(g1_p.reshape(N, DH), src3, dst3)
    w2p = jnp.zeros((DH, DH), jnp.float32).at[:, :NCLS].set(W2)
    m2 = jnp.kron(jnp.eye(GP, dtype=jnp.float32), w2p)
    b2p = jnp.zeros((DH,), jnp.float32).at[:NCLS].set(b2)
    b2t = jnp.tile(b2p, GP).reshape(1, GP * DH)
    g2_p = _tc2(p1.reshape(NC, NP, GP * DH), g1_p, dinv_p, m2, b2t)

    p2 = _sc_agg(g2_p.reshape(N, DH), src3, dst3)
    s0 = jnp.kron(jnp.eye(GP, dtype=jnp.float32), jnp.ones((DH, DH), jnp.float32))
    y_p = _tc3(p2.reshape(NC, NP, GP * DH), g2_p, dinv_p, s0)
    return y_p.reshape(N, DH)[:, :NCLS]


# speedup vs baseline: 74.5095x; 1.0939x over previous
import functools

import jax
import jax.numpy as jnp
from jax import lax
from jax.experimental import pallas as pl
from jax.experimental.pallas import tpu as pltpu
from jax.experimental.pallas import tpu_sc as plsc

N = 10000
E = 320000
DF = 128
DH = 16
NCLS = 10

NC = 2
NS = 16
NW = NC * NS
EPT = E // NW
B = 125
NB = EPT // B
RPT = 624
TAIL0 = RPT * NS
TAILN = N - TAIL0

@functools.cache
def _sc_kernels():
    mesh = plsc.VectorSubcoreMesh(core_axis_name="c", subcore_axis_name="s")
    params = pltpu.CompilerParams(use_tc_tiling_on_sc=False)

    @functools.partial(
        pl.kernel,
        out_type=jax.ShapeDtypeStruct((NC, N, DH), jnp.float32),
        mesh=mesh,
        compiler_params=params,
        scratch_types=[
            pltpu.VMEM((NB, B), jnp.int32),
            pltpu.VMEM((B, DH), jnp.float32),
            pltpu.VMEM_SHARED((N, DH), jnp.float32),
        ],
    )
    def sc_degree(edge_hbm, ones_hbm, out_hbm, dst_v, ones_v, acc):
        c = lax.axis_index("c")
        s = lax.axis_index("s")
        wid = s * NC + c
        pltpu.sync_copy(ones_hbm.at[pl.ds(s * RPT, RPT)], acc.at[pl.ds(s * RPT, RPT)])

        @pl.when(s == NS - 1)
        def _():
            pltpu.sync_copy(ones_hbm.at[pl.ds(TAIL0, TAILN)], acc.at[pl.ds(TAIL0, TAILN)])

        pltpu.sync_copy(ones_hbm.at[pl.ds(0, B)], ones_v)
        pltpu.sync_copy(edge_hbm.at[1, wid], dst_v)
        plsc.subcore_barrier()

        def body(j, carry):
            pltpu.sync_copy(ones_v, acc.at[dst_v.at[j]], add=True)
            return carry

        lax.fori_loop(0, NB, body, 0)
        plsc.subcore_barrier()
        pltpu.sync_copy(acc.at[pl.ds(s * RPT, RPT)], out_hbm.at[c, pl.ds(s * RPT, RPT)])

        @pl.when(s == NS - 1)
        def _():
            pltpu.sync_copy(acc.at[pl.ds(TAIL0, TAILN)], out_hbm.at[c, pl.ds(TAIL0, TAILN)])

    @functools.partial(
        pl.kernel,
        out_type=jax.ShapeDtypeStruct((NC, N, DH), jnp.float32),
        mesh=mesh,
        compiler_params=params,
        scratch_types=[
            pltpu.VMEM((NB, B), jnp.int32),
            pltpu.VMEM((NB, B), jnp.int32),
            pltpu.VMEM((4, B, DH), jnp.float32),
            pltpu.VMEM_SHARED((N, DH), jnp.float32),
            [pltpu.SemaphoreType.DMA] * 4,
        ],
    )
    def sc_agg(g_hbm, edge_hbm, out_hbm, src_v, dst_v, msg, acc, sems):
        c = lax.axis_index("c")
        s = lax.axis_index("s")
        wid = s * NC + c
        pltpu.sync_copy(g_hbm.at[pl.ds(s * RPT, RPT)], acc.at[pl.ds(s * RPT, RPT)])

        @pl.when(s == NS - 1)
        def _():
            pltpu.sync_copy(g_hbm.at[pl.ds(TAIL0, TAILN)], acc.at[pl.ds(TAIL0, TAILN)])

        pltpu.sync_copy(edge_hbm.at[0, wid], src_v)
        pltpu.sync_copy(edge_hbm.at[1, wid], dst_v)
        plsc.subcore_barrier()

        for k in range(3):
            pltpu.async_copy(g_hbm.at[src_v.at[k]], msg.at[k], sems[k])
        ngrp = NB // 4

        def body(g, carry):
            j0 = 4 * g
            for k in range(4):
                j = j0 + k
                kn = (k + 3) % 4
                pltpu.make_async_copy(g_hbm.at[src_v.at[j]], msg.at[k], sems[k]).wait()

                @pl.when(j + 3 < NB)
                def _():
                    pltpu.async_copy(g_hbm.at[src_v.at[j + 3]], msg.at[kn], sems[kn])

                pltpu.sync_copy(msg.at[k], acc.at[dst_v.at[j]], add=True)
            return carry

        lax.fori_loop(0, ngrp, body, 0)
        plsc.subcore_barrier()
        pltpu.sync_copy(acc.at[pl.ds(s * RPT, RPT)], out_hbm.at[c, pl.ds(s * RPT, RPT)])

        @pl.when(s == NS - 1)
        def _():
            pltpu.sync_copy(acc.at[pl.ds(TAIL0, TAILN)], out_hbm.at[c, pl.ds(TAIL0, TAILN)])

    return sc_degree, sc_agg


NP = N // 8
GP = 8


def _tc1a_body(x3_ref, w1_ref, b1_ref, h_ref):
    w1 = w1_ref[...]
    h = jnp.concatenate(
        [
            jnp.dot(x3_ref[:, j, :], w1, preferred_element_type=jnp.float32)
            for j in range(GP)
        ],
        axis=1,
    )
    h_ref[...] = h + b1_ref[...]


def _tc1b_body(degp_ref, h_ref, g_ref, dinv_ref):
    deg = degp_ref[0] + degp_ref[1] - 1.0
    dinv = lax.rsqrt(deg)
    g_ref[...] = h_ref[...] * dinv
    dinv_ref[...] = dinv


def _tc2_body(p_ref, g1_ref, dinv_ref, m2_ref, b2_ref, g2_ref):
    agg = p_ref[0] + p_ref[1] - g1_ref[...]
    out1 = jnp.maximum(dinv_ref[...] * agg, 0.0)
    h2 = jnp.dot(out1, m2_ref[...], preferred_element_type=jnp.float32)
    g2_ref[...] = (h2 + b2_ref[...]) * dinv_ref[...]


def _tc3_body(p_ref, g2_ref, dinv_ref, s0_ref, y_ref):
    out2 = dinv_ref[...] * (p_ref[0] + p_ref[1] - g2_ref[...])
    lane = lax.broadcasted_iota(jnp.int32, (NP, GP * DH), 1)
    mask = (lane % DH) < NCLS
    zm = jnp.where(mask, out2, 0.0)
    s0 = s0_ref[...]
    mean = jnp.dot(zm, s0, preferred_element_type=jnp.float32) * (1.0 / NCLS)
    e = jnp.where(mask, jnp.exp(out2 - mean), 0.0)
    tot = jnp.dot(e, s0, preferred_element_type=jnp.float32)
    y_ref[...] = e / tot


_full = lambda shape: pl.BlockSpec(shape, lambda: tuple(0 for _ in shape))

_tc1a = pl.pallas_call(
    _tc1a_body,
    in_specs=[
        _full((NP, GP, DF)),
        _full((DF, DH)),
        _full((1, GP * DH)),
    ],
    out_specs=_full((NP, GP * DH)),
    out_shape=jax.ShapeDtypeStruct((NP, GP * DH), jnp.float32),
)

_tc1b = pl.pallas_call(
    _tc1b_body,
    in_specs=[
        _full((NC, NP, GP * DH)),
        _full((NP, GP * DH)),
    ],
    out_specs=[_full((NP, GP * DH)), _full((NP, GP * DH))],
    out_shape=[
        jax.ShapeDtypeStruct((NP, GP * DH), jnp.float32),
        jax.ShapeDtypeStruct((NP, GP * DH), jnp.float32),
    ],
)

_tc2 = pl.pallas_call(
    _tc2_body,
    in_specs=[
        _full((NC, NP, GP * DH)),
        _full((NP, GP * DH)),
        _full((NP, GP * DH)),
        _full((GP * DH, GP * DH)),
        _full((1, GP * DH)),
    ],
    out_specs=_full((NP, GP * DH)),
    out_shape=jax.ShapeDtypeStruct((NP, GP * DH), jnp.float32),
)

_tc3 = pl.pallas_call(
    _tc3_body,
    in_specs=[
        _full((NC, NP, GP * DH)),
        _full((NP, GP * DH)),
        _full((NP, GP * DH)),
        _full((GP * DH, GP * DH)),
    ],
    out_specs=_full((NP, GP * DH)),
    out_shape=jax.ShapeDtypeStruct((NP, GP * DH), jnp.float32),
)


@jax.jit
def kernel(x, edge_index, W1, b1, W2, b2):
    ei = edge_index.astype(jnp.int32).reshape(2, NW, NB, B)

    ones_nd = jnp.ones((N, DH), jnp.float32)

    _sc_degree, _sc_agg = _sc_kernels()
    degp = _sc_degree(ei, ones_nd)

    x3 = x.reshape(NP, GP, DF)
    b1t = jnp.tile(b1, GP).reshape(1, GP * DH)
    h1_p = _tc1a(x3, W1, b1t)
    g1_p, dinv_p = _tc1b(degp.reshape(NC, NP, GP * DH), h1_p)

    p1 = _sc_agg(g1_p.reshape(N, DH), ei)
    w2p = jnp.zeros((DH, DH), jnp.float32).at[:, :NCLS].set(W2)
    m2 = jnp.kron(jnp.eye(GP, dtype=jnp.float32), w2p)
    b2p = jnp.zeros((DH,), jnp.float32).at[:NCLS].set(b2)
    b2t = jnp.tile(b2p, GP).reshape(1, GP * DH)
    g2_p = _tc2(p1.reshape(NC, NP, GP * DH), g1_p, dinv_p, m2, b2t)

    p2 = _sc_agg(g2_p.reshape(N, DH), ei)
    s0 = jnp.kron(jnp.eye(GP, dtype=jnp.float32), jnp.ones((DH, DH), jnp.float32))
    y_p = _tc3(p2.reshape(NC, NP, GP * DH), g2_p, dinv_p, s0)
    return y_p.reshape(N, DH)[:, :NCLS]


# speedup vs baseline: 87.1595x; 1.1698x over previous
import functools

import jax
import jax.numpy as jnp
from jax import lax
from jax.experimental import pallas as pl
from jax.experimental.pallas import tpu as pltpu
from jax.experimental.pallas import tpu_sc as plsc

N = 10000
E = 320000
DF = 128
DH = 16
NCLS = 10

NC = 2
NS = 16
NW = NC * NS
EPT = E // NW
B = 125
NB = EPT // B
RPT = 624
TAIL0 = RPT * NS
TAILN = N - TAIL0

@functools.cache
def _sc_kernels():
    mesh = plsc.VectorSubcoreMesh(core_axis_name="c", subcore_axis_name="s")
    params = pltpu.CompilerParams(use_tc_tiling_on_sc=False)

    @functools.partial(
        pl.kernel,
        out_type=jax.ShapeDtypeStruct((NC, N, DH), jnp.float32),
        mesh=mesh,
        compiler_params=params,
        scratch_types=[
            pltpu.VMEM((NB, B), jnp.int32),
            pltpu.VMEM((B, DH), jnp.float32),
            pltpu.VMEM_SHARED((N, DH), jnp.float32),
        ],
    )
    def sc_degree(edge_hbm, ones_hbm, out_hbm, dst_v, ones_v, acc):
        c = lax.axis_index("c")
        s = lax.axis_index("s")
        wid = s * NC + c
        pltpu.sync_copy(ones_hbm.at[pl.ds(s * RPT, RPT)], acc.at[pl.ds(s * RPT, RPT)])

        @pl.when(s == NS - 1)
        def _():
            pltpu.sync_copy(ones_hbm.at[pl.ds(TAIL0, TAILN)], acc.at[pl.ds(TAIL0, TAILN)])

        pltpu.sync_copy(ones_hbm.at[pl.ds(0, B)], ones_v)
        pltpu.sync_copy(edge_hbm.at[1, wid], dst_v)
        plsc.subcore_barrier()

        def body(j, carry):
            pltpu.sync_copy(ones_v, acc.at[dst_v.at[j]], add=True)
            return carry

        lax.fori_loop(0, NB, body, 0)
        plsc.subcore_barrier()
        pltpu.sync_copy(acc.at[pl.ds(s * RPT, RPT)], out_hbm.at[c, pl.ds(s * RPT, RPT)])

        @pl.when(s == NS - 1)
        def _():
            pltpu.sync_copy(acc.at[pl.ds(TAIL0, TAILN)], out_hbm.at[c, pl.ds(TAIL0, TAILN)])

    @functools.partial(
        pl.kernel,
        out_type=jax.ShapeDtypeStruct((NC, N, DH), jnp.float32),
        mesh=mesh,
        compiler_params=params,
        scratch_types=[
            pltpu.VMEM((NB, B), jnp.int32),
            pltpu.VMEM((NB, B), jnp.int32),
            pltpu.VMEM((8, B, DH), jnp.float32),
            pltpu.VMEM_SHARED((N, DH), jnp.float32),
            [pltpu.SemaphoreType.DMA] * 8,
        ],
    )
    def sc_agg(g_hbm, edge_hbm, out_hbm, src_v, dst_v, msg, acc, sems):
        c = lax.axis_index("c")
        s = lax.axis_index("s")
        wid = s * NC + c
        pltpu.sync_copy(g_hbm.at[pl.ds(s * RPT, RPT)], acc.at[pl.ds(s * RPT, RPT)])

        @pl.when(s == NS - 1)
        def _():
            pltpu.sync_copy(g_hbm.at[pl.ds(TAIL0, TAILN)], acc.at[pl.ds(TAIL0, TAILN)])

        pltpu.sync_copy(edge_hbm.at[0, wid], src_v)
        pltpu.sync_copy(edge_hbm.at[1, wid], dst_v)
        plsc.subcore_barrier()

        for k in range(7):
            pltpu.async_copy(g_hbm.at[src_v.at[k]], msg.at[k], sems[k])
        ngrp = NB // 8

        def body(g, carry):
            j0 = 8 * g
            for k in range(8):
                j = j0 + k
                kn = (k + 7) % 8
                pltpu.make_async_copy(g_hbm.at[src_v.at[j]], msg.at[k], sems[k]).wait()

                @pl.when(j + 7 < NB)
                def _():
                    pltpu.async_copy(g_hbm.at[src_v.at[j + 7]], msg.at[kn], sems[kn])

                pltpu.sync_copy(msg.at[k], acc.at[dst_v.at[j]], add=True)
            return carry

        lax.fori_loop(0, ngrp, body, 0)
        plsc.subcore_barrier()
        pltpu.sync_copy(acc.at[pl.ds(s * RPT, RPT)], out_hbm.at[c, pl.ds(s * RPT, RPT)])

        @pl.when(s == NS - 1)
        def _():
            pltpu.sync_copy(acc.at[pl.ds(TAIL0, TAILN)], out_hbm.at[c, pl.ds(TAIL0, TAILN)])

    return sc_degree, sc_agg


NP = N // 8
GP = 8


def _tc1a_body(x3_ref, w1_ref, b1_ref, h_ref):
    w1 = w1_ref[...]
    h = jnp.concatenate(
        [
            jnp.dot(x3_ref[:, j, :], w1, preferred_element_type=jnp.float32)
            for j in range(GP)
        ],
        axis=1,
    )
    h_ref[...] = h + b1_ref[...]


def _tc1b_body(degp_ref, h_ref, g_ref, dinv_ref):
    deg = degp_ref[0] + degp_ref[1] - 1.0
    dinv = lax.rsqrt(deg)
    g_ref[...] = h_ref[...] * dinv
    dinv_ref[...] = dinv


def _tc2_body(p_ref, g1_ref, dinv_ref, m2_ref, b2_ref, g2_ref):
    agg = p_ref[0] + p_ref[1] - g1_ref[...]
    out1 = jnp.maximum(dinv_ref[...] * agg, 0.0)
    h2 = jnp.dot(out1, m2_ref[...], preferred_element_type=jnp.float32)
    g2_ref[...] = (h2 + b2_ref[...]) * dinv_ref[...]


def _tc3_body(p_ref, g2_ref, dinv_ref, s0_ref, y_ref):
    out2 = dinv_ref[...] * (p_ref[0] + p_ref[1] - g2_ref[...])
    lane = lax.broadcasted_iota(jnp.int32, (NP, GP * DH), 1)
    mask = (lane % DH) < NCLS
    zm = jnp.where(mask, out2, 0.0)
    s0 = s0_ref[...]
    mean = jnp.dot(zm, s0, preferred_element_type=jnp.float32) * (1.0 / NCLS)
    e = jnp.where(mask, jnp.exp(out2 - mean), 0.0)
    tot = jnp.dot(e, s0, preferred_element_type=jnp.float32)
    y_ref[...] = e / tot


_full = lambda shape: pl.BlockSpec(shape, lambda: tuple(0 for _ in shape))

_tc1a = pl.pallas_call(
    _tc1a_body,
    in_specs=[
        _full((NP, GP, DF)),
        _full((DF, DH)),
        _full((1, GP * DH)),
    ],
    out_specs=_full((NP, GP * DH)),
    out_shape=jax.ShapeDtypeStruct((NP, GP * DH), jnp.float32),
)

_tc1b = pl.pallas_call(
    _tc1b_body,
    in_specs=[
        _full((NC, NP, GP * DH)),
        _full((NP, GP * DH)),
    ],
    out_specs=[_full((NP, GP * DH)), _full((NP, GP * DH))],
    out_shape=[
        jax.ShapeDtypeStruct((NP, GP * DH), jnp.float32),
        jax.ShapeDtypeStruct((NP, GP * DH), jnp.float32),
    ],
)

_tc2 = pl.pallas_call(
    _tc2_body,
    in_specs=[
        _full((NC, NP, GP * DH)),
        _full((NP, GP * DH)),
        _full((NP, GP * DH)),
        _full((GP * DH, GP * DH)),
        _full((1, GP * DH)),
    ],
    out_specs=_full((NP, GP * DH)),
    out_shape=jax.ShapeDtypeStruct((NP, GP * DH), jnp.float32),
)

_tc3 = pl.pallas_call(
    _tc3_body,
    in_specs=[
        _full((NC, NP, GP * DH)),
        _full((NP, GP * DH)),
        _full((NP, GP * DH)),
        _full((GP * DH, GP * DH)),
    ],
    out_specs=_full((NP, GP * DH)),
    out_shape=jax.ShapeDtypeStruct((NP, GP * DH), jnp.float32),
)


@jax.jit
def kernel(x, edge_index, W1, b1, W2, b2):
    ei = edge_index.astype(jnp.int32).reshape(2, NW, NB, B)

    ones_nd = jnp.ones((N, DH), jnp.float32)

    _sc_degree, _sc_agg = _sc_kernels()
    degp = _sc_degree(ei, ones_nd)

    x3 = x.reshape(NP, GP, DF)
    b1t = jnp.tile(b1, GP).reshape(1, GP * DH)
    h1_p = _tc1a(x3, W1, b1t)
    g1_p, dinv_p = _tc1b(degp.reshape(NC, NP, GP * DH), h1_p)

    p1 = _sc_agg(g1_p.reshape(N, DH), ei)
    w2p = jnp.zeros((DH, DH), jnp.float32).at[:, :NCLS].set(W2)
    m2 = jnp.kron(jnp.eye(GP, dtype=jnp.float32), w2p)
    b2p = jnp.zeros((DH,), jnp.float32).at[:NCLS].set(b2)
    b2t = jnp.tile(b2p, GP).reshape(1, GP * DH)
    g2_p = _tc2(p1.reshape(NC, NP, GP * DH), g1_p, dinv_p, m2, b2t)

    p2 = _sc_agg(g2_p.reshape(N, DH), ei)
    s0 = jnp.kron(jnp.eye(GP, dtype=jnp.float32), jnp.ones((DH, DH), jnp.float32))
    y_p = _tc3(p2.reshape(NC, NP, GP * DH), g2_p, dinv_p, s0)
    return y_p.reshape(N, DH)[:, :NCLS]


# speedup vs baseline: 87.2524x; 1.0011x over previous
import functools

import jax
import jax.numpy as jnp
from jax import lax
from jax.experimental import pallas as pl
from jax.experimental.pallas import tpu as pltpu
from jax.experimental.pallas import tpu_sc as plsc

N = 10000
E = 320000
DF = 128
DH = 16
NCLS = 10

NC = 2
NS = 16
NW = NC * NS
EPT = E // NW
B = 125
NB = EPT // B
RPT = 624
TAIL0 = RPT * NS
TAILN = N - TAIL0

@functools.cache
def _sc_kernels():
    mesh = plsc.VectorSubcoreMesh(core_axis_name="c", subcore_axis_name="s")
    params = pltpu.CompilerParams(use_tc_tiling_on_sc=False)

    @functools.partial(
        pl.kernel,
        out_type=jax.ShapeDtypeStruct((NC, N, DH), jnp.float32),
        mesh=mesh,
        compiler_params=params,
        scratch_types=[
            pltpu.VMEM((NB, B), jnp.int32),
            pltpu.VMEM((B, DH), jnp.float32),
            pltpu.VMEM_SHARED((N, DH), jnp.float32),
        ],
    )
    def sc_degree(edge_hbm, ones_hbm, out_hbm, dst_v, ones_v, acc):
        c = lax.axis_index("c")
        s = lax.axis_index("s")
        wid = s * NC + c
        pltpu.sync_copy(ones_hbm.at[pl.ds(s * RPT, RPT)], acc.at[pl.ds(s * RPT, RPT)])

        @pl.when(s == NS - 1)
        def _():
            pltpu.sync_copy(ones_hbm.at[pl.ds(TAIL0, TAILN)], acc.at[pl.ds(TAIL0, TAILN)])

        pltpu.sync_copy(ones_hbm.at[pl.ds(0, B)], ones_v)
        pltpu.sync_copy(edge_hbm.at[1, wid], dst_v)
        plsc.subcore_barrier()

        def body(j, carry):
            pltpu.sync_copy(ones_v, acc.at[dst_v.at[j]], add=True)
            return carry

        lax.fori_loop(0, NB, body, 0)
        plsc.subcore_barrier()
        pltpu.sync_copy(acc.at[pl.ds(s * RPT, RPT)], out_hbm.at[c, pl.ds(s * RPT, RPT)])

        @pl.when(s == NS - 1)
        def _():
            pltpu.sync_copy(acc.at[pl.ds(TAIL0, TAILN)], out_hbm.at[c, pl.ds(TAIL0, TAILN)])

    @functools.partial(
        pl.kernel,
        out_type=jax.ShapeDtypeStruct((NC, N, DH), jnp.float32),
        mesh=mesh,
        compiler_params=params,
        scratch_types=[
            pltpu.VMEM((NB, B), jnp.int32),
            pltpu.VMEM((NB, B), jnp.int32),
            pltpu.VMEM((8, B, DH), jnp.float32),
            pltpu.VMEM_SHARED((N, DH), jnp.float32),
            [pltpu.SemaphoreType.DMA] * 8,
            [pltpu.SemaphoreType.DMA] * 8,
        ],
    )
    def sc_agg(g_hbm, edge_hbm, out_hbm, src_v, dst_v, msg, acc, sems, ssems):
        c = lax.axis_index("c")
        s = lax.axis_index("s")
        wid = s * NC + c
        pltpu.sync_copy(g_hbm.at[pl.ds(s * RPT, RPT)], acc.at[pl.ds(s * RPT, RPT)])

        @pl.when(s == NS - 1)
        def _():
            pltpu.sync_copy(g_hbm.at[pl.ds(TAIL0, TAILN)], acc.at[pl.ds(TAIL0, TAILN)])

        pltpu.sync_copy(edge_hbm.at[0, wid], src_v)
        pltpu.sync_copy(edge_hbm.at[1, wid], dst_v)
        plsc.subcore_barrier()

        for k in range(7):
            pltpu.async_copy(g_hbm.at[src_v.at[k]], msg.at[k], sems[k])
        ngrp = NB // 8

        def body(g, carry):
            j0 = 8 * g
            for k in range(8):
                j = j0 + k
                kn = (k + 7) % 8
                pltpu.make_async_copy(g_hbm.at[src_v.at[j]], msg.at[k], sems[k]).wait()

                @pl.when(j + 7 < NB)
                def _():
                    @pl.when(j >= 1)
                    def _():
                        pltpu.make_async_copy(
                            msg.at[kn], acc.at[dst_v.at[j]], ssems[kn]
                        ).wait()

                    pltpu.async_copy(g_hbm.at[src_v.at[j + 7]], msg.at[kn], sems[kn])

                pltpu.async_copy(msg.at[k], acc.at[dst_v.at[j]], ssems[k], add=True)
            return carry

        lax.fori_loop(0, ngrp, body, 0)
        for k in range(8):
            pltpu.make_async_copy(msg.at[k], acc.at[dst_v.at[0]], ssems[k]).wait()
        plsc.subcore_barrier()
        pltpu.sync_copy(acc.at[pl.ds(s * RPT, RPT)], out_hbm.at[c, pl.ds(s * RPT, RPT)])

        @pl.when(s == NS - 1)
        def _():
            pltpu.sync_copy(acc.at[pl.ds(TAIL0, TAILN)], out_hbm.at[c, pl.ds(TAIL0, TAILN)])

    return sc_degree, sc_agg


NP = N // 8
GP = 8


def _tc1a_body(x3_ref, w1_ref, b1_ref, h_ref):
    w1 = w1_ref[...]
    h = jnp.concatenate(
        [
            jnp.dot(x3_ref[:, j, :], w1, preferred_element_type=jnp.float32)
            for j in range(GP)
        ],
        axis=1,
    )
    h_ref[...] = h + b1_ref[...]


def _tc1b_body(degp_ref, h_ref, g_ref, dinv_ref):
    deg = degp_ref[0] + degp_ref[1] - 1.0
    dinv = lax.rsqrt(deg)
    g_ref[...] = h_ref[...] * dinv
    dinv_ref[...] = dinv


def _tc2_body(p_ref, g1_ref, dinv_ref, m2_ref, b2_ref, g2_ref):
    agg = p_ref[0] + p_ref[1] - g1_ref[...]
    out1 = jnp.maximum(dinv_ref[...] * agg, 0.0)
    h2 = jnp.dot(out1, m2_ref[...], preferred_element_type=jnp.float32)
    g2_ref[...] = (h2 + b2_ref[...]) * dinv_ref[...]


def _tc3_body(p_ref, g2_ref, dinv_ref, s0_ref, y_ref):
    out2 = dinv_ref[...] * (p_ref[0] + p_ref[1] - g2_ref[...])
    lane = lax.broadcasted_iota(jnp.int32, (NP, GP * DH), 1)
    mask = (lane % DH) < NCLS
    zm = jnp.where(mask, out2, 0.0)
    s0 = s0_ref[...]
    mean = jnp.dot(zm, s0, preferred_element_type=jnp.float32) * (1.0 / NCLS)
    e = jnp.where(mask, jnp.exp(out2 - mean), 0.0)
    tot = jnp.dot(e, s0, preferred_element_type=jnp.float32)
    y_ref[...] = e / tot


_full = lambda shape: pl.BlockSpec(shape, lambda: tuple(0 for _ in shape))

_tc1a = pl.pallas_call(
    _tc1a_body,
    in_specs=[
        _full((NP, GP, DF)),
        _full((DF, DH)),
        _full((1, GP * DH)),
    ],
    out_specs=_full((NP, GP * DH)),
    out_shape=jax.ShapeDtypeStruct((NP, GP * DH), jnp.float32),
)

_tc1b = pl.pallas_call(
    _tc1b_body,
    in_specs=[
        _full((NC, NP, GP * DH)),
        _full((NP, GP * DH)),
    ],
    out_specs=[_full((NP, GP * DH)), _full((NP, GP * DH))],
    out_shape=[
        jax.ShapeDtypeStruct((NP, GP * DH), jnp.float32),
        jax.ShapeDtypeStruct((NP, GP * DH), jnp.float32),
    ],
)

_tc2 = pl.pallas_call(
    _tc2_body,
    in_specs=[
        _full((NC, NP, GP * DH)),
        _full((NP, GP * DH)),
        _full((NP, GP * DH)),
        _full((GP * DH, GP * DH)),
        _full((1, GP * DH)),
    ],
    out_specs=_full((NP, GP * DH)),
    out_shape=jax.ShapeDtypeStruct((NP, GP * DH), jnp.float32),
)

_tc3 = pl.pallas_call(
    _tc3_body,
    in_specs=[
        _full((NC, NP, GP * DH)),
        _full((NP, GP * DH)),
        _full((NP, GP * DH)),
        _full((GP * DH, GP * DH)),
    ],
    out_specs=_full((NP, GP * DH)),
    out_shape=jax.ShapeDtypeStruct((NP, GP * DH), jnp.float32),
)


@jax.jit
def kernel(x, edge_index, W1, b1, W2, b2):
    ei = edge_index.astype(jnp.int32).reshape(2, NW, NB, B)

    ones_nd = jnp.ones((N, DH), jnp.float32)

    _sc_degree, _sc_agg = _sc_kernels()
    degp = _sc_degree(ei, ones_nd)

    x3 = x.reshape(NP, GP, DF)
    b1t = jnp.tile(b1, GP).reshape(1, GP * DH)
    h1_p = _tc1a(x3, W1, b1t)
    g1_p, dinv_p = _tc1b(degp.reshape(NC, NP, GP * DH), h1_p)

    p1 = _sc_agg(g1_p.reshape(N, DH), ei)
    w2p = jnp.zeros((DH, DH), jnp.float32).at[:, :NCLS].set(W2)
    m2 = jnp.kron(jnp.eye(GP, dtype=jnp.float32), w2p)
    b2p = jnp.zeros((DH,), jnp.float32).at[:NCLS].set(b2)
    b2t = jnp.tile(b2p, GP).reshape(1, GP * DH)
    g2_p = _tc2(p1.reshape(NC, NP, GP * DH), g1_p, dinv_p, m2, b2t)

    p2 = _sc_agg(g2_p.reshape(N, DH), ei)
    s0 = jnp.kron(jnp.eye(GP, dtype=jnp.float32), jnp.ones((DH, DH), jnp.float32))
    y_p = _tc3(p2.reshape(NC, NP, GP * DH), g2_p, dinv_p, s0)
    return y_p.reshape(N, DH)[:, :NCLS]
